# baseline (speedup 1.0000x reference)
import functools

import jax
import jax.numpy as jnp
from jax import lax
from jax.experimental import pallas as pl
from jax.experimental.pallas import tpu as pltpu
from jax.experimental.pallas import tpu_sc as plsc

EMB_DIM = 512
HALF_DIM = EMB_DIM // 2
POS_DIM = 25
ENT_DIM = 512

BATCH = 16
NCAND = 128
NENT = 30

B_TOTAL = BATCH * NCAND * NENT
KCH = 8
BCH = BATCH // KCH
B_CHUNK = B_TOTAL // KCH

NW = 32
B_PER_W = B_CHUNK // NW
CH = 120
N_CHUNK = B_PER_W // CH

RBLK = NENT * NCAND
NLANE = 16


def _make_sc_gather():
    mesh = plsc.VectorSubcoreMesh(core_axis_name="c", subcore_axis_name="s")

    @functools.partial(
        pl.kernel,
        mesh=mesh,
        out_type=(
            jax.ShapeDtypeStruct((B_CHUNK, HALF_DIM), jnp.float32),
            jax.ShapeDtypeStruct((B_CHUNK,), jnp.int32),
        ),
        scratch_types=[
            pltpu.VMEM((B_PER_W,), jnp.int32),
            pltpu.VMEM((B_PER_W,), jnp.int32),
            pltpu.VMEM((CH, EMB_DIM), jnp.float32),
            pltpu.VMEM((CH, EMB_DIM), jnp.float32),
            pltpu.SemaphoreType.DMA,
            pltpu.SemaphoreType.DMA,
            pltpu.SemaphoreType.DMA,
            pltpu.SemaphoreType.DMA,
            pltpu.SemaphoreType.DMA,
        ],
    )
    def sc_gather(table_hbm, pidx_hbm, ids_hbm, g_hbm, pid_hbm,
                  idx_all, pid_all, rows0, rows1,
                  gsem0, gsem1, wsem0, wsem1, psem):
        nc = 2
        wid = lax.axis_index("s") * nc + lax.axis_index("c")
        base = wid * B_PER_W
        rows = (rows0, rows1)
        gsem = (gsem0, gsem1)
        wsem = (wsem0, wsem1)

        pltpu.sync_copy(ids_hbm.at[pl.ds(base, B_PER_W)], idx_all)
        pid_cp = pltpu.async_copy(pidx_hbm.at[idx_all], pid_all, psem)

        def g_issue(c, b):
            return pltpu.async_copy(
                table_hbm.at[idx_all.at[pl.ds(c * CH, CH)]], rows[b], gsem[b])

        def g_wait(b):
            pltpu.make_async_copy(
                table_hbm.at[idx_all.at[pl.ds(0, CH)]], rows[b], gsem[b]
            ).wait()

        def w_issue(c, b):
            return pltpu.async_copy(
                rows[b].at[:, pl.ds(0, HALF_DIM)],
                g_hbm.at[pl.ds(base + c * CH, CH)], wsem[b])

        def w_wait(b):
            pltpu.make_async_copy(
                rows[b].at[:, pl.ds(0, HALF_DIM)],
                g_hbm.at[pl.ds(base, CH)], wsem[b]).wait()

        def pack_rows(b):
            def row_body(i, carry):
                for gidx in range(EMB_DIM // 32):
                    ua = lax.bitcast_convert_type(
                        rows[b][i, pl.ds(32 * gidx, NLANE)], jnp.int32)
                    uc = lax.bitcast_convert_type(
                        rows[b][i, pl.ds(32 * gidx + NLANE, NLANE)], jnp.int32)
                    ra = ua + jnp.int32(0x8000)
                    rc = uc + jnp.int32(0x8000)
                    word = (lax.shift_right_logical(ra, 16)
                            | (rc & jnp.int32(-65536)))
                    rows[b][i, pl.ds(NLANE * gidx, NLANE)] = (
                        lax.bitcast_convert_type(word, jnp.float32))
                return carry

            lax.fori_loop(0, CH, row_body, 0)

        g_issue(0, 0)

        def body(c2, carry):
            for b in range(2):
                c = c2 * 2 + b
                nb = 1 - b

                @pl.when(c + 1 < N_CHUNK)
                def _():
                    @pl.when(c >= 1)
                    def _():
                        w_wait(nb)

                    g_issue(c + 1, nb)

                g_wait(b)
                pack_rows(b)
                w_issue(c, b)
            return carry

        lax.fori_loop(0, N_CHUNK // 2, body, 0)
        w_wait(0)
        w_wait(1)
        pid_cp.wait()
        pltpu.sync_copy(pid_all, pid_hbm.at[pl.ds(base, B_PER_W)])

    return sc_gather


_sc_gather = _make_sc_gather()


def _tc_body(g_ref, w_ref, pos_ref, b_ref, welo_ref, wehi_ref, pid_ref,
             out_ref):
    u = lax.bitcast_convert_type(
        g_ref[...].reshape(RBLK, HALF_DIM), jnp.int32)
    lo = lax.bitcast_convert_type(u << 16, jnp.float32)
    hi = lax.bitcast_convert_type(u & jnp.int32(-65536), jnp.float32)
    wp = w_ref[:, EMB_DIM:]
    pos16 = pos_ref[...]
    bias16 = lax.dot_general(
        pos16, wp, (((1,), (1,)), ((), ())),
        preferred_element_type=jnp.float32) + b_ref[...]
    pid = pid_ref[0, 0, :]
    onehot_t = (lax.broadcasted_iota(jnp.int32, (16, RBLK), 0)
                == pid[None, :]).astype(jnp.float32)
    out = lax.dot_general(
        lo, welo_ref[...], (((1,), (1,)), ((), ())),
        preferred_element_type=jnp.float32)
    out = out + lax.dot_general(
        hi, wehi_ref[...], (((1,), (1,)), ((), ())),
        preferred_element_type=jnp.float32)
    out = out + lax.dot_general(
        onehot_t, bias16, (((0,), (0,)), ((), ())),
        preferred_element_type=jnp.float32)
    out_ref[...] = out.reshape(1, NENT, NCAND, ENT_DIM)


def _tc_body_chained(g_ref, w_ref, pos_ref, b_ref, welo_ref, wehi_ref,
                     pid_ref, prev_ref, out_ref):
    del prev_ref
    _tc_body(g_ref, w_ref, pos_ref, b_ref, welo_ref, wehi_ref, pid_ref,
             out_ref)


def _tc_project_chunk(k, g3, w, pos16, b2, welo, wehi, pid3, prev):
    in_specs = [
        pl.BlockSpec((NENT, NCAND, HALF_DIM), lambda i: (i, 0, 0)),
        pl.BlockSpec((ENT_DIM, EMB_DIM + POS_DIM), lambda i: (0, 0)),
        pl.BlockSpec((16, POS_DIM), lambda i: (0, 0)),
        pl.BlockSpec((1, ENT_DIM), lambda i: (0, 0)),
        pl.BlockSpec((ENT_DIM, HALF_DIM), lambda i: (0, 0)),
        pl.BlockSpec((ENT_DIM, HALF_DIM), lambda i: (0, 0)),
        pl.BlockSpec((1, 1, RBLK), lambda i: (i, 0, 0)),
    ]
    args = (g3, w, pos16, b2, welo, wehi, pid3)
    body = _tc_body
    aliases = {}
    if prev is not None:
        in_specs.append(pl.BlockSpec(memory_space=pltpu.MemorySpace.HBM))
        args = args + (prev,)
        body = _tc_body_chained
        aliases = {7: 0}
    return pl.pallas_call(
        body,
        grid=(BCH,),
        in_specs=in_specs,
        out_specs=pl.BlockSpec((1, NENT, NCAND, ENT_DIM),
                               lambda i, k=k: (k * BCH + i, 0, 0, 0)),
        out_shape=jax.ShapeDtypeStruct((BATCH, NENT, NCAND, ENT_DIM),
                                       jnp.float32),
        input_output_aliases=aliases,
    )(*args)


def kernel(entity_ids, entity_table, pos_table, entity_id_to_pos_index, W, b):
    ids = entity_ids.transpose(0, 2, 1).reshape(KCH, B_CHUNK).astype(jnp.int32)
    pidx = entity_id_to_pos_index.astype(jnp.int32)
    pos16 = pos_table[:16]
    b2 = b.reshape(1, ENT_DIM)
    cols = jnp.arange(HALF_DIM)
    flo = 32 * (cols // NLANE) + cols % NLANE
    we = W[:, :EMB_DIM]
    welo = jnp.take(we, flo, axis=1)
    wehi = jnp.take(we, flo + NLANE, axis=1)

    gathered = [_sc_gather(entity_table, pidx, ids[0]),
                _sc_gather(entity_table, pidx, ids[1])]
    out = None
    for k in range(KCH):
        g, pid = gathered[k]
        g3 = g.reshape(BCH * NENT, NCAND, HALF_DIM)
        pid3 = pid.reshape(BCH, 1, RBLK)
        out = _tc_project_chunk(k, g3, W, pos16, b2, welo, wehi, pid3, out)
        if k + 2 < KCH:
            ids_n, out = lax.optimization_barrier((ids[k + 2], out))
            gathered.append(_sc_gather(entity_table, pidx, ids_n))
    return out.transpose(0, 2, 1, 3)


# speedup vs baseline: 1.0866x; 1.0866x over previous
import functools

import jax
import jax.numpy as jnp
from jax import lax
from jax.experimental import pallas as pl
from jax.experimental.pallas import tpu as pltpu
from jax.experimental.pallas import tpu_sc as plsc

EMB_DIM = 512
HALF_DIM = EMB_DIM // 2
POS_DIM = 25
ENT_DIM = 512

BATCH = 16
NCAND = 128
NENT = 30

B_TOTAL = BATCH * NCAND * NENT
KCH = 4
BCH = BATCH // KCH
B_CHUNK = B_TOTAL // KCH

NW = 32
B_PER_W = B_CHUNK // NW
CH = 120
N_CHUNK = B_PER_W // CH

RBLK = NENT * NCAND
NLANE = 16


def _make_sc_gather():
    mesh = plsc.VectorSubcoreMesh(core_axis_name="c", subcore_axis_name="s")

    @functools.partial(
        pl.kernel,
        mesh=mesh,
        out_type=(
            jax.ShapeDtypeStruct((B_CHUNK, HALF_DIM), jnp.float32),
            jax.ShapeDtypeStruct((B_CHUNK,), jnp.int32),
        ),
        scratch_types=[
            pltpu.VMEM((B_PER_W,), jnp.int32),
            pltpu.VMEM((B_PER_W,), jnp.int32),
            pltpu.VMEM((CH, EMB_DIM), jnp.float32),
            pltpu.VMEM((CH, EMB_DIM), jnp.float32),
            pltpu.SemaphoreType.DMA,
            pltpu.SemaphoreType.DMA,
            pltpu.SemaphoreType.DMA,
            pltpu.SemaphoreType.DMA,
            pltpu.SemaphoreType.DMA,
        ],
    )
    def sc_gather(table_hbm, pidx_hbm, ids_hbm, g_hbm, pid_hbm,
                  idx_all, pid_all, rows0, rows1,
                  gsem0, gsem1, wsem0, wsem1, psem):
        nc = 2
        wid = lax.axis_index("s") * nc + lax.axis_index("c")
        base = wid * B_PER_W
        rows = (rows0, rows1)
        gsem = (gsem0, gsem1)
        wsem = (wsem0, wsem1)

        pltpu.sync_copy(ids_hbm.at[pl.ds(base, B_PER_W)], idx_all)
        pid_cp = pltpu.async_copy(pidx_hbm.at[idx_all], pid_all, psem)

        def g_issue(c, b):
            return pltpu.async_copy(
                table_hbm.at[idx_all.at[pl.ds(c * CH, CH)]], rows[b], gsem[b])

        def g_wait(b):
            pltpu.make_async_copy(
                table_hbm.at[idx_all.at[pl.ds(0, CH)]], rows[b], gsem[b]
            ).wait()

        def w_issue(c, b):
            return pltpu.async_copy(
                rows[b].at[:, pl.ds(0, HALF_DIM)],
                g_hbm.at[pl.ds(base + c * CH, CH)], wsem[b])

        def w_wait(b):
            pltpu.make_async_copy(
                rows[b].at[:, pl.ds(0, HALF_DIM)],
                g_hbm.at[pl.ds(base, CH)], wsem[b]).wait()

        def pack_rows(b):
            def row_body(i, carry):
                for gidx in range(EMB_DIM // 32):
                    ua = lax.bitcast_convert_type(
                        rows[b][i, pl.ds(32 * gidx, NLANE)], jnp.int32)
                    uc = lax.bitcast_convert_type(
                        rows[b][i, pl.ds(32 * gidx + NLANE, NLANE)], jnp.int32)
                    ra = ua + jnp.int32(0x8000)
                    rc = uc + jnp.int32(0x8000)
                    word = (lax.shift_right_logical(ra, 16)
                            | (rc & jnp.int32(-65536)))
                    rows[b][i, pl.ds(NLANE * gidx, NLANE)] = (
                        lax.bitcast_convert_type(word, jnp.float32))
                return carry

            lax.fori_loop(0, CH, row_body, 0)

        g_issue(0, 0)

        def body(c2, carry):
            for b in range(2):
                c = c2 * 2 + b
                nb = 1 - b

                @pl.when(c + 1 < N_CHUNK)
                def _():
                    @pl.when(c >= 1)
                    def _():
                        w_wait(nb)

                    g_issue(c + 1, nb)

                g_wait(b)
                pack_rows(b)
                w_issue(c, b)
            return carry

        lax.fori_loop(0, N_CHUNK // 2, body, 0)
        w_wait(0)
        w_wait(1)
        pid_cp.wait()
        pltpu.sync_copy(pid_all, pid_hbm.at[pl.ds(base, B_PER_W)])

    return sc_gather


_sc_gather = _make_sc_gather()


def _tc_body(g_ref, w_ref, pos_ref, b_ref, welo_ref, wehi_ref, pid_ref,
             out_ref):
    u = lax.bitcast_convert_type(
        g_ref[...].reshape(RBLK, HALF_DIM), jnp.int32)
    lo = lax.bitcast_convert_type(u << 16, jnp.float32)
    hi = lax.bitcast_convert_type(u & jnp.int32(-65536), jnp.float32)
    wp = w_ref[:, EMB_DIM:]
    pos16 = pos_ref[...]
    bias16 = lax.dot_general(
        pos16, wp, (((1,), (1,)), ((), ())),
        preferred_element_type=jnp.float32) + b_ref[...]
    pid = pid_ref[0, 0, :]
    onehot_t = (lax.broadcasted_iota(jnp.int32, (16, RBLK), 0)
                == pid[None, :]).astype(jnp.float32)
    out = lax.dot_general(
        lo, welo_ref[...], (((1,), (1,)), ((), ())),
        preferred_element_type=jnp.float32)
    out = out + lax.dot_general(
        hi, wehi_ref[...], (((1,), (1,)), ((), ())),
        preferred_element_type=jnp.float32)
    out = out + lax.dot_general(
        onehot_t, bias16, (((0,), (0,)), ((), ())),
        preferred_element_type=jnp.float32)
    out_ref[...] = out.reshape(1, NENT, NCAND, ENT_DIM)


def _tc_body_chained(g_ref, w_ref, pos_ref, b_ref, welo_ref, wehi_ref,
                     pid_ref, prev_ref, out_ref):
    del prev_ref
    _tc_body(g_ref, w_ref, pos_ref, b_ref, welo_ref, wehi_ref, pid_ref,
             out_ref)


def _tc_project_chunk(k, g3, w, pos16, b2, welo, wehi, pid3, prev):
    in_specs = [
        pl.BlockSpec((NENT, NCAND, HALF_DIM), lambda i: (i, 0, 0)),
        pl.BlockSpec((ENT_DIM, EMB_DIM + POS_DIM), lambda i: (0, 0)),
        pl.BlockSpec((16, POS_DIM), lambda i: (0, 0)),
        pl.BlockSpec((1, ENT_DIM), lambda i: (0, 0)),
        pl.BlockSpec((ENT_DIM, HALF_DIM), lambda i: (0, 0)),
        pl.BlockSpec((ENT_DIM, HALF_DIM), lambda i: (0, 0)),
        pl.BlockSpec((1, 1, RBLK), lambda i: (i, 0, 0)),
    ]
    args = (g3, w, pos16, b2, welo, wehi, pid3)
    body = _tc_body
    aliases = {}
    if prev is not None:
        in_specs.append(pl.BlockSpec(memory_space=pltpu.MemorySpace.HBM))
        args = args + (prev,)
        body = _tc_body_chained
        aliases = {7: 0}
    return pl.pallas_call(
        body,
        grid=(BCH,),
        in_specs=in_specs,
        out_specs=pl.BlockSpec((1, NENT, NCAND, ENT_DIM),
                               lambda i, k=k: (k * BCH + i, 0, 0, 0)),
        out_shape=jax.ShapeDtypeStruct((BATCH, NENT, NCAND, ENT_DIM),
                                       jnp.float32),
        input_output_aliases=aliases,
    )(*args)


def kernel(entity_ids, entity_table, pos_table, entity_id_to_pos_index, W, b):
    ids = entity_ids.transpose(0, 2, 1).reshape(KCH, B_CHUNK).astype(jnp.int32)
    pidx = entity_id_to_pos_index.astype(jnp.int32)
    pos16 = pos_table[:16]
    b2 = b.reshape(1, ENT_DIM)
    cols = jnp.arange(HALF_DIM)
    flo = 32 * (cols // NLANE) + cols % NLANE
    we = W[:, :EMB_DIM]
    welo = jnp.take(we, flo, axis=1)
    wehi = jnp.take(we, flo + NLANE, axis=1)

    gathered = [_sc_gather(entity_table, pidx, ids[0]),
                _sc_gather(entity_table, pidx, ids[1])]
    out = None
    for k in range(KCH):
        g, pid = gathered[k]
        g3 = g.reshape(BCH * NENT, NCAND, HALF_DIM)
        pid3 = pid.reshape(BCH, 1, RBLK)
        out = _tc_project_chunk(k, g3, W, pos16, b2, welo, wehi, pid3, out)
        if k + 2 < KCH:
            ids_n, out = lax.optimization_barrier((ids[k + 2], out))
            gathered.append(_sc_gather(entity_table, pidx, ids_n))
    return out.transpose(0, 2, 1, 3)
